# baseline (speedup 1.0000x reference)
import jax
import jax.numpy as jnp
from jax.experimental import pallas as pl
from jax.experimental.pallas import tpu as pltpu

CLASSES = 10199
BATCH = 16384
ROWS_PER_BLOCK = 512


def _onehot_block(cap_ref, out_ref):
    cap = cap_ref[:, :]
    cols = jax.lax.broadcasted_iota(jnp.int32, (ROWS_PER_BLOCK, CLASSES), 1)
    out_ref[:, :] = (cols == cap).astype(jnp.float32)


def kernel(obj, caption, puzzle):
    cap = caption[0][:, None]
    grid = BATCH // ROWS_PER_BLOCK
    return pl.pallas_call(
        _onehot_block,
        grid=(grid,),
        in_specs=[pl.BlockSpec((ROWS_PER_BLOCK, 1), lambda i: (i, 0))],
        out_specs=pl.BlockSpec((ROWS_PER_BLOCK, CLASSES), lambda i: (i, 0)),
        out_shape=jax.ShapeDtypeStruct((BATCH, CLASSES), jnp.float32),
        compiler_params=pltpu.CompilerParams(
            dimension_semantics=("parallel",),
        ),
    )(cap)


# speedup vs baseline: 1.0032x; 1.0032x over previous
import jax
import jax.numpy as jnp
from jax.experimental import pallas as pl
from jax.experimental.pallas import tpu as pltpu

CLASSES = 10199
BATCH = 16384
ROWS_PER_BLOCK = 128
NSLOTS = 4
GRID = BATCH // ROWS_PER_BLOCK


def _onehot_block(cap_ref, out_ref, buf, sems):
    i = pl.program_id(0)
    slot = jax.lax.rem(i, NSLOTS)

    @pl.when(i >= NSLOTS)
    def _wait_prev():
        row0 = (i - NSLOTS) * ROWS_PER_BLOCK
        pltpu.make_async_copy(
            buf.at[slot], out_ref.at[pl.ds(row0, ROWS_PER_BLOCK), :], sems.at[slot]
        ).wait()

    cap = cap_ref[:, :]
    cols = jax.lax.broadcasted_iota(jnp.int32, (ROWS_PER_BLOCK, CLASSES), 1)
    buf[slot] = (cols == cap).astype(jnp.float32)

    pltpu.make_async_copy(
        buf.at[slot], out_ref.at[pl.ds(i * ROWS_PER_BLOCK, ROWS_PER_BLOCK), :],
        sems.at[slot],
    ).start()

    @pl.when(i == GRID - 1)
    def _drain():
        for s in range(NSLOTS):
            row0 = (GRID - NSLOTS + s) * ROWS_PER_BLOCK
            pltpu.make_async_copy(
                buf.at[(GRID - NSLOTS + s) % NSLOTS],
                out_ref.at[pl.ds(row0, ROWS_PER_BLOCK), :],
                sems.at[(GRID - NSLOTS + s) % NSLOTS],
            ).wait()


def kernel(obj, caption, puzzle):
    cap = caption[0][:, None]
    return pl.pallas_call(
        _onehot_block,
        grid=(GRID,),
        in_specs=[pl.BlockSpec((ROWS_PER_BLOCK, 1), lambda i: (i, 0))],
        out_specs=pl.BlockSpec(memory_space=pl.ANY),
        out_shape=jax.ShapeDtypeStruct((BATCH, CLASSES), jnp.float32),
        scratch_shapes=[
            pltpu.VMEM((NSLOTS, ROWS_PER_BLOCK, CLASSES), jnp.float32),
            pltpu.SemaphoreType.DMA((NSLOTS,)),
        ],
    )(cap)
